# baseline (device time: 1276914 ns/iter reference)
import jax
import jax.numpy as jnp
from jax import lax
from jax.experimental import pallas as pl
from jax.experimental.pallas import tpu as pltpu

N_DEV = 32
N_HOPS = N_DEV - 1
N_LAYERS = 3


def kernel(x, Win0, Wout0, Win1, Wout1, Win2, Wout2):
    b, d = x.shape
    _, hs = Win0.shape

    def body(x_ref, win0_ref, wout0_ref, win1_ref, wout1_ref,
             win2_ref, wout2_ref, out_ref,
             win_buf, wout_buf, win_ssem, win_rsem, wout_ssem, wout_rsem):
        my = lax.axis_index("i")
        left = lax.rem(my + (N_DEV - 1), N_DEV)
        right = lax.rem(my + 1, N_DEV)

        barrier = pltpu.get_barrier_semaphore()
        for nbr in (left, right):
            pl.semaphore_signal(barrier, inc=1, device_id=(nbr,),
                                device_id_type=pl.DeviceIdType.MESH)
        pl.semaphore_wait(barrier, 2)

        def contrib(x_val, w_in, w_out):
            h = jnp.maximum(
                jnp.dot(x_val, w_in, preferred_element_type=jnp.float32), 0.0)
            return jnp.dot(h, w_out, preferred_element_type=jnp.float32)

        def one_layer(l, x_val, win_ref, wout_ref):
            win_buf[l, 0] = win_ref[...]
            wout_buf[l, 0] = wout_ref[...]
            acc = contrib(x_val, win_buf[l, 0], wout_buf[l, 0])

            def hop(s, acc):
                r = 1 - s
                cp_w = pltpu.make_async_remote_copy(
                    src_ref=win_buf.at[l, s], dst_ref=win_buf.at[l, r],
                    send_sem=win_ssem.at[l, s], recv_sem=win_rsem.at[l, r],
                    device_id=(right,), device_id_type=pl.DeviceIdType.MESH)
                cp_o = pltpu.make_async_remote_copy(
                    src_ref=wout_buf.at[l, s], dst_ref=wout_buf.at[l, r],
                    send_sem=wout_ssem.at[l, s], recv_sem=wout_rsem.at[l, r],
                    device_id=(right,), device_id_type=pl.DeviceIdType.MESH)
                cp_w.start()
                cp_o.start()
                cp_w.wait()
                cp_o.wait()
                return acc + contrib(x_val, win_buf[l, r], wout_buf[l, r])

            def pair(i, acc):
                return hop(1, hop(0, acc))

            acc = lax.fori_loop(0, N_HOPS // 2, pair, acc)
            if N_HOPS % 2:
                acc = hop(0, acc)
            return acc

        xc = x_ref[...]
        xc = one_layer(0, xc, win0_ref, wout0_ref)
        xc = one_layer(1, xc, win1_ref, wout1_ref)
        xc = one_layer(2, xc, win2_ref, wout2_ref)
        out_ref[...] = xc

    vmem = pl.BlockSpec(memory_space=pltpu.VMEM)
    return pl.pallas_call(
        body,
        out_shape=jax.ShapeDtypeStruct((b, d), jnp.float32),
        in_specs=[vmem] * 7,
        out_specs=vmem,
        scratch_shapes=[
            pltpu.VMEM((N_LAYERS, 2, d, hs), jnp.float32),
            pltpu.VMEM((N_LAYERS, 2, hs, d), jnp.float32),
            pltpu.SemaphoreType.DMA((N_LAYERS, 2)),
            pltpu.SemaphoreType.DMA((N_LAYERS, 2)),
            pltpu.SemaphoreType.DMA((N_LAYERS, 2)),
            pltpu.SemaphoreType.DMA((N_LAYERS, 2)),
        ],
        compiler_params=pltpu.CompilerParams(collective_id=0),
    )(x, Win0, Wout0, Win1, Wout1, Win2, Wout2)


# device time: 637237 ns/iter; 2.0038x vs baseline; 2.0038x over previous
import jax
import jax.numpy as jnp
from jax import lax
from jax.experimental import pallas as pl
from jax.experimental.pallas import tpu as pltpu

N_DEV = 32
N_LAYERS = 3
FWD_HOPS = N_DEV // 2
BWD_HOPS = N_DEV // 2 - 1


def kernel(x, Win0, Wout0, Win1, Wout1, Win2, Wout2):
    b, d = x.shape
    _, hs = Win0.shape
    bf16 = jnp.bfloat16

    def body(x_ref, win0_ref, wout0_ref, win1_ref, wout1_ref,
             win2_ref, wout2_ref, out_ref,
             fwin, fwout, bwin, bwout,
             fwin_ss, fwin_rs, fwout_ss, fwout_rs,
             bwin_ss, bwin_rs, bwout_ss, bwout_rs):
        my = lax.axis_index("i")
        left = lax.rem(my + (N_DEV - 1), N_DEV)
        right = lax.rem(my + 1, N_DEV)

        barrier = pltpu.get_barrier_semaphore()
        for nbr in (left, right):
            pl.semaphore_signal(barrier, inc=1, device_id=(nbr,),
                                device_id_type=pl.DeviceIdType.MESH)
        pl.semaphore_wait(barrier, 2)

        def descs(l, s, fwd):
            tgt = right if fwd else left
            wbuf, obuf = (fwin, fwout) if fwd else (bwin, bwout)
            wss, wrs = (fwin_ss, fwin_rs) if fwd else (bwin_ss, bwin_rs)
            oss, ors = (fwout_ss, fwout_rs) if fwd else (bwout_ss, bwout_rs)
            r = 1 - s
            cw = pltpu.make_async_remote_copy(
                src_ref=wbuf.at[l, s], dst_ref=wbuf.at[l, r],
                send_sem=wss.at[l, s], recv_sem=wrs.at[l, r],
                device_id=(tgt,), device_id_type=pl.DeviceIdType.MESH)
            co = pltpu.make_async_remote_copy(
                src_ref=obuf.at[l, s], dst_ref=obuf.at[l, r],
                send_sem=oss.at[l, s], recv_sem=ors.at[l, r],
                device_id=(tgt,), device_id_type=pl.DeviceIdType.MESH)
            return cw, co

        def hop(l, s, fwd=True, bwd=True):
            ds = []
            if fwd:
                ds += list(descs(l, s, True))
            if bwd:
                ds += list(descs(l, s, False))
            for c in ds:
                c.start()
            for c in ds:
                c.wait()

        def contrib(x_bf, l, slot, fwd):
            wbuf, obuf = (fwin, fwout) if fwd else (bwin, bwout)
            h = jnp.maximum(
                jnp.dot(x_bf, wbuf[l, slot],
                        preferred_element_type=jnp.float32), 0.0)
            return jnp.dot(h.astype(bf16), obuf[l, slot],
                           preferred_element_type=jnp.float32)

        def one_layer(l, xc, win_ref, wout_ref):
            x_bf = xc.astype(bf16)
            win_bf = win_ref[...].astype(bf16)
            wout_bf = wout_ref[...].astype(bf16)
            fwin[l, 0] = win_bf
            fwout[l, 0] = wout_bf
            bwin[l, 0] = win_bf
            bwout[l, 0] = wout_bf
            acc = contrib(x_bf, l, 0, True)

            def pair(k, acc):
                hop(l, 0)
                acc = acc + contrib(x_bf, l, 1, True)
                acc = acc + contrib(x_bf, l, 1, False)
                hop(l, 1)
                acc = acc + contrib(x_bf, l, 0, True)
                acc = acc + contrib(x_bf, l, 0, False)
                return acc

            acc = lax.fori_loop(0, 7, pair, acc)
            hop(l, 0)
            acc = acc + contrib(x_bf, l, 1, True)
            acc = acc + contrib(x_bf, l, 1, False)
            hop(l, 1, fwd=True, bwd=False)
            acc = acc + contrib(x_bf, l, 0, True)
            return acc

        xc = x_ref[...]
        xc = one_layer(0, xc, win0_ref, wout0_ref)
        xc = one_layer(1, xc, win1_ref, wout1_ref)
        xc = one_layer(2, xc, win2_ref, wout2_ref)
        out_ref[...] = xc

    vmem = pl.BlockSpec(memory_space=pltpu.VMEM)
    dma2 = pltpu.SemaphoreType.DMA((N_LAYERS, 2))
    return pl.pallas_call(
        body,
        out_shape=jax.ShapeDtypeStruct((b, d), jnp.float32),
        in_specs=[vmem] * 7,
        out_specs=vmem,
        scratch_shapes=[
            pltpu.VMEM((N_LAYERS, 2, d, hs), bf16),
            pltpu.VMEM((N_LAYERS, 2, hs, d), bf16),
            pltpu.VMEM((N_LAYERS, 2, d, hs), bf16),
            pltpu.VMEM((N_LAYERS, 2, hs, d), bf16),
            dma2, dma2, dma2, dma2,
            dma2, dma2, dma2, dma2,
        ],
        compiler_params=pltpu.CompilerParams(collective_id=0),
    )(x, Win0, Wout0, Win1, Wout1, Win2, Wout2)


# device time: 626264 ns/iter; 2.0389x vs baseline; 1.0175x over previous
import jax
import jax.numpy as jnp
from jax import lax
from jax.experimental import pallas as pl
from jax.experimental.pallas import tpu as pltpu

N_DEV = 32
N_LAYERS = 3
FWD_HOPS = N_DEV // 2
BWD_HOPS = N_DEV // 2 - 1


def kernel(x, Win0, Wout0, Win1, Wout1, Win2, Wout2):
    b, d = x.shape
    _, hs = Win0.shape
    bf16 = jnp.bfloat16

    def body(x_ref, win0_ref, wout0_ref, win1_ref, wout1_ref,
             win2_ref, wout2_ref, out_ref,
             fwin, fwout, bwin, bwout,
             fwin_ss, fwin_rs, fwout_ss, fwout_rs,
             bwin_ss, bwin_rs, bwout_ss, bwout_rs):
        my = lax.axis_index("i")
        left = lax.rem(my + (N_DEV - 1), N_DEV)
        right = lax.rem(my + 1, N_DEV)

        barrier = pltpu.get_barrier_semaphore()
        for nbr in (left, right):
            pl.semaphore_signal(barrier, inc=1, device_id=(nbr,),
                                device_id_type=pl.DeviceIdType.MESH)
        pl.semaphore_wait(barrier, 2)

        def descs(l, s, fwd):
            tgt = right if fwd else left
            wbuf, obuf = (fwin, fwout) if fwd else (bwin, bwout)
            wss, wrs = (fwin_ss, fwin_rs) if fwd else (bwin_ss, bwin_rs)
            oss, ors = (fwout_ss, fwout_rs) if fwd else (bwout_ss, bwout_rs)
            r = 1 - s
            cw = pltpu.make_async_remote_copy(
                src_ref=wbuf.at[l, s], dst_ref=wbuf.at[l, r],
                send_sem=wss.at[l, s], recv_sem=wrs.at[l, r],
                device_id=(tgt,), device_id_type=pl.DeviceIdType.MESH)
            co = pltpu.make_async_remote_copy(
                src_ref=obuf.at[l, s], dst_ref=obuf.at[l, r],
                send_sem=oss.at[l, s], recv_sem=ors.at[l, r],
                device_id=(tgt,), device_id_type=pl.DeviceIdType.MESH)
            return cw, co

        def hop_start(l, s, fwd=True, bwd=True):
            ds = []
            if fwd:
                ds += list(descs(l, s, True))
            if bwd:
                ds += list(descs(l, s, False))
            for c in ds:
                c.start()
            return ds

        def hop_wait(ds):
            for c in ds:
                c.wait()

        def contrib(x_bf, l, slot, fwd):
            wbuf, obuf = (fwin, fwout) if fwd else (bwin, bwout)
            h = jnp.maximum(
                jnp.dot(x_bf, wbuf[l, slot],
                        preferred_element_type=jnp.float32), 0.0)
            return jnp.dot(h.astype(bf16), obuf[l, slot],
                           preferred_element_type=jnp.float32)

        def one_layer(l, xc, win_ref, wout_ref):
            x_bf = xc.astype(bf16)
            win_bf = win_ref[...].astype(bf16)
            wout_bf = wout_ref[...].astype(bf16)
            fwin[l, 0] = win_bf
            fwout[l, 0] = wout_bf
            bwin[l, 0] = win_bf
            bwout[l, 0] = wout_bf
            d1 = hop_start(l, 0)
            acc = contrib(x_bf, l, 0, True)
            hop_wait(d1)
            def pair(k, acc):
                de = hop_start(l, 1)
                acc = acc + contrib(x_bf, l, 1, True)
                acc = acc + contrib(x_bf, l, 1, False)
                hop_wait(de)
                do = hop_start(l, 0)
                acc = acc + contrib(x_bf, l, 0, True)
                acc = acc + contrib(x_bf, l, 0, False)
                hop_wait(do)
                return acc

            acc = lax.fori_loop(0, 7, pair, acc)
            d16 = hop_start(l, 1, fwd=True, bwd=False)
            acc = acc + contrib(x_bf, l, 1, True)
            acc = acc + contrib(x_bf, l, 1, False)
            hop_wait(d16)
            acc = acc + contrib(x_bf, l, 0, True)
            return acc

        xc = x_ref[...]
        xc = one_layer(0, xc, win0_ref, wout0_ref)
        xc = one_layer(1, xc, win1_ref, wout1_ref)
        xc = one_layer(2, xc, win2_ref, wout2_ref)
        out_ref[...] = xc

    vmem = pl.BlockSpec(memory_space=pltpu.VMEM)
    dma2 = pltpu.SemaphoreType.DMA((N_LAYERS, 2))
    return pl.pallas_call(
        body,
        out_shape=jax.ShapeDtypeStruct((b, d), jnp.float32),
        in_specs=[vmem] * 7,
        out_specs=vmem,
        scratch_shapes=[
            pltpu.VMEM((N_LAYERS, 2, d, hs), bf16),
            pltpu.VMEM((N_LAYERS, 2, hs, d), bf16),
            pltpu.VMEM((N_LAYERS, 2, d, hs), bf16),
            pltpu.VMEM((N_LAYERS, 2, hs, d), bf16),
            dma2, dma2, dma2, dma2,
            dma2, dma2, dma2, dma2,
        ],
        compiler_params=pltpu.CompilerParams(collective_id=0),
    )(x, Win0, Wout0, Win1, Wout1, Win2, Wout2)
